# baseline (speedup 1.0000x reference)
import functools

import jax
import jax.numpy as jnp
from jax import lax
from jax.experimental import pallas as pl
from jax.experimental.pallas import tpu as pltpu
from jax.experimental.pallas import tpu_sc as plsc

HIDDEN = 1024
SEQ = 512
EPS = 1e-12

NUM_WORKERS = 32
CHUNK = 16
NBUF = 4
CHUNK_SIZES = (2048, 8192, 8192, 8192, 4096, 2048)


def _sc_gather(ids_flat, word_table, row_off, n):
    b_per_w = n // NUM_WORKERS
    nchunks = b_per_w // CHUNK
    mesh = plsc.VectorSubcoreMesh(core_axis_name="c", subcore_axis_name="s")

    @functools.partial(
        pl.kernel,
        out_type=jax.ShapeDtypeStruct((n, HIDDEN), jnp.float32),
        mesh=mesh,
        scratch_types=(
            [pltpu.VMEM((b_per_w,), jnp.int32)]
            + [pltpu.VMEM((CHUNK, HIDDEN), jnp.float32) for _ in range(NBUF)]
            + [pltpu.SemaphoreType.DMA for _ in range(2 * NBUF)]
        ),
    )
    def gather_kernel(table_hbm, ids_hbm, out_hbm, idx_v, *scratch):
        bufs = scratch[:NBUF]
        gsems = scratch[NBUF:2 * NBUF]
        wsems = scratch[2 * NBUF:]
        wid = lax.axis_index("s") * 2 + lax.axis_index("c")
        base = wid * b_per_w

        pltpu.sync_copy(ids_hbm.at[pl.ds(row_off + base, b_per_w)], idx_v)

        def start_gather(b, c):
            pltpu.async_copy(
                table_hbm.at[idx_v.at[pl.ds(c * CHUNK, CHUNK)]],
                bufs[b], gsems[b])

        def wait_gather(b, c):
            pltpu.make_async_copy(
                table_hbm.at[idx_v.at[pl.ds(c * CHUNK, CHUNK)]],
                bufs[b], gsems[b]).wait()

        def start_write(b, c):
            pltpu.async_copy(
                bufs[b], out_hbm.at[pl.ds(base + c * CHUNK, CHUNK)],
                wsems[b])

        def wait_write(b, c):
            pltpu.make_async_copy(
                bufs[b], out_hbm.at[pl.ds(base + c * CHUNK, CHUNK)],
                wsems[b]).wait()

        for b in range(NBUF):
            start_gather(b, b)

        @pl.loop(0, nchunks, step=NBUF)
        def _(c0):
            for b in range(NBUF):
                c = c0 + b
                wait_gather(b, c)
                start_write(b, c)

                @pl.when(c0 + NBUF < nchunks)
                def _():
                    wait_write(b, c)
                    start_gather(b, c + NBUF)

        for b in range(NBUF):
            wait_write(b, nchunks - NBUF + b)

    return gather_kernel(word_table, ids_flat)


LN_R = 512
LN_NB = 4


def _ln_math(emb, gamma, beta):
    mean = jnp.mean(emb, axis=-1, keepdims=True)
    ctr = emb - mean
    var = jnp.mean(ctr * ctr, axis=-1, keepdims=True)
    return (ctr * lax.rsqrt(var + EPS)) * gamma + beta


def _make_ln_body(rows, row_off):
    nblk = rows // LN_R

    def body(emb_hbm, pos_ref, gamma_ref, beta_ref, out_hbm, *scr):
        ibufs = scr[:LN_NB]
        obufs = scr[LN_NB:2 * LN_NB]
        isems = scr[2 * LN_NB:3 * LN_NB]
        osems = scr[3 * LN_NB:]

        def in_copy(b, c):
            return pltpu.make_async_copy(
                emb_hbm.at[pl.ds(c * LN_R, LN_R), :], ibufs[b], isems[b])

        def out_copy(b, c):
            return pltpu.make_async_copy(
                obufs[b], out_hbm.at[pl.ds(row_off + c * LN_R, LN_R), :],
                osems[b])

        for b in range(LN_NB):
            in_copy(b, b).start()

        @pl.loop(0, nblk, step=LN_NB)
        def _(c0):
            for b in range(LN_NB):
                c = c0 + b
                in_copy(b, c).wait()

                @pl.when(c0 >= LN_NB)
                def _():
                    out_copy(b, c - LN_NB).wait()

                obufs[b][...] = _ln_math(ibufs[b][...] + pos_ref[...],
                                         gamma_ref[...], beta_ref[...])
                out_copy(b, c).start()

                @pl.when(c0 + LN_NB < nblk)
                def _():
                    in_copy(b, c + LN_NB).start()

        for b in range(LN_NB):
            out_copy(b, nblk - LN_NB + b).wait()

    return body


def _tc_layernorm_chunk(out_buf, gathered, pos_table, gamma2, beta2, row_off):
    rows = gathered.shape[0]
    body = _make_ln_body(rows, row_off)
    data_specs = [
        pl.BlockSpec(memory_space=pl.ANY),
        pl.BlockSpec(memory_space=pltpu.VMEM),
        pl.BlockSpec(memory_space=pltpu.VMEM),
        pl.BlockSpec(memory_space=pltpu.VMEM),
    ]
    scratch = (
        [pltpu.VMEM((LN_R, HIDDEN), jnp.float32) for _ in range(2 * LN_NB)]
        + [pltpu.SemaphoreType.DMA for _ in range(2 * LN_NB)]
    )
    n_total = sum(CHUNK_SIZES)
    if out_buf is None:
        return pl.pallas_call(
            body,
            in_specs=data_specs,
            out_specs=pl.BlockSpec(memory_space=pl.ANY),
            out_shape=jax.ShapeDtypeStruct((n_total, HIDDEN), jnp.float32),
            scratch_shapes=scratch,
        )(gathered, pos_table, gamma2, beta2)
    return pl.pallas_call(
        lambda alias_ref, *a: body(*a),
        in_specs=[pl.BlockSpec(memory_space=pl.ANY)] + data_specs,
        out_specs=pl.BlockSpec(memory_space=pl.ANY),
        out_shape=jax.ShapeDtypeStruct((n_total, HIDDEN), jnp.float32),
        input_output_aliases={0: 0},
        scratch_shapes=scratch,
    )(out_buf, gathered, pos_table, gamma2, beta2)


def kernel(input_ids, word_table, pos_table, gamma, beta):
    B, S = input_ids.shape
    n = B * S
    ids_flat = input_ids.reshape(n).astype(jnp.int32)
    gamma2 = gamma.reshape(1, HIDDEN)
    beta2 = beta.reshape(1, HIDDEN)

    offs = [0]
    for sz in CHUNK_SIZES:
        offs.append(offs[-1] + sz)
    assert offs[-1] == n

    gathered = [
        _sc_gather(ids_flat, word_table, offs[k], sz)
        for k, sz in enumerate(CHUNK_SIZES)
    ]

    out = None
    for k in range(len(CHUNK_SIZES)):
        out = _tc_layernorm_chunk(out, gathered[k], pos_table, gamma2, beta2,
                                  offs[k])
    return out.reshape(B, S, HIDDEN)


# speedup vs baseline: 1.0258x; 1.0258x over previous
import functools

import jax
import jax.numpy as jnp
from jax import lax
from jax.experimental import pallas as pl
from jax.experimental.pallas import tpu as pltpu
from jax.experimental.pallas import tpu_sc as plsc

HIDDEN = 1024
SEQ = 512
EPS = 1e-12

NUM_WORKERS = 32
CHUNK = 16
NBUF = 4
CHUNK_SIZES = (8192, 8192, 8192, 8192)


def _sc_gather(ids_flat, word_table, row_off, n):
    b_per_w = n // NUM_WORKERS
    nchunks = b_per_w // CHUNK
    mesh = plsc.VectorSubcoreMesh(core_axis_name="c", subcore_axis_name="s")

    @functools.partial(
        pl.kernel,
        out_type=jax.ShapeDtypeStruct((n, HIDDEN), jnp.float32),
        mesh=mesh,
        scratch_types=(
            [pltpu.VMEM((b_per_w,), jnp.int32)]
            + [pltpu.VMEM((CHUNK, HIDDEN), jnp.float32) for _ in range(NBUF)]
            + [pltpu.SemaphoreType.DMA for _ in range(2 * NBUF)]
        ),
    )
    def gather_kernel(table_hbm, ids_hbm, out_hbm, idx_v, *scratch):
        bufs = scratch[:NBUF]
        gsems = scratch[NBUF:2 * NBUF]
        wsems = scratch[2 * NBUF:]
        wid = lax.axis_index("s") * 2 + lax.axis_index("c")
        base = wid * b_per_w

        pltpu.sync_copy(ids_hbm.at[pl.ds(row_off + base, b_per_w)], idx_v)

        def start_gather(b, c):
            pltpu.async_copy(
                table_hbm.at[idx_v.at[pl.ds(c * CHUNK, CHUNK)]],
                bufs[b], gsems[b])

        def wait_gather(b, c):
            pltpu.make_async_copy(
                table_hbm.at[idx_v.at[pl.ds(c * CHUNK, CHUNK)]],
                bufs[b], gsems[b]).wait()

        def start_write(b, c):
            pltpu.async_copy(
                bufs[b], out_hbm.at[pl.ds(base + c * CHUNK, CHUNK)],
                wsems[b])

        def wait_write(b, c):
            pltpu.make_async_copy(
                bufs[b], out_hbm.at[pl.ds(base + c * CHUNK, CHUNK)],
                wsems[b]).wait()

        for b in range(NBUF):
            start_gather(b, b)

        @pl.loop(0, nchunks, step=NBUF)
        def _(c0):
            for b in range(NBUF):
                c = c0 + b
                wait_gather(b, c)
                start_write(b, c)

                @pl.when(c0 + NBUF < nchunks)
                def _():
                    wait_write(b, c)
                    start_gather(b, c + NBUF)

        for b in range(NBUF):
            wait_write(b, nchunks - NBUF + b)

    return gather_kernel(word_table, ids_flat)


LN_R = 512
LN_NB = 4


def _ln_math(emb, gamma, beta):
    mean = jnp.mean(emb, axis=-1, keepdims=True)
    ctr = emb - mean
    var = jnp.mean(ctr * ctr, axis=-1, keepdims=True)
    return (ctr * lax.rsqrt(var + EPS)) * gamma + beta


def _make_ln_body(rows, row_off):
    nblk = rows // LN_R

    def body(emb_hbm, pos_ref, gamma_ref, beta_ref, out_hbm, *scr):
        ibufs = scr[:LN_NB]
        obufs = scr[LN_NB:2 * LN_NB]
        isems = scr[2 * LN_NB:3 * LN_NB]
        osems = scr[3 * LN_NB:]

        def in_copy(b, c):
            return pltpu.make_async_copy(
                emb_hbm.at[pl.ds(c * LN_R, LN_R), :], ibufs[b], isems[b])

        def out_copy(b, c):
            return pltpu.make_async_copy(
                obufs[b], out_hbm.at[pl.ds(row_off + c * LN_R, LN_R), :],
                osems[b])

        for b in range(LN_NB):
            in_copy(b, b).start()

        @pl.loop(0, nblk, step=LN_NB)
        def _(c0):
            for b in range(LN_NB):
                c = c0 + b
                in_copy(b, c).wait()

                @pl.when(c0 >= LN_NB)
                def _():
                    out_copy(b, c - LN_NB).wait()

                obufs[b][...] = _ln_math(ibufs[b][...] + pos_ref[...],
                                         gamma_ref[...], beta_ref[...])
                out_copy(b, c).start()

                @pl.when(c0 + LN_NB < nblk)
                def _():
                    in_copy(b, c + LN_NB).start()

        for b in range(LN_NB):
            out_copy(b, nblk - LN_NB + b).wait()

    return body


def _tc_layernorm_chunk(out_buf, gathered, pos_table, gamma2, beta2, row_off):
    rows = gathered.shape[0]
    body = _make_ln_body(rows, row_off)
    data_specs = [
        pl.BlockSpec(memory_space=pl.ANY),
        pl.BlockSpec(memory_space=pltpu.VMEM),
        pl.BlockSpec(memory_space=pltpu.VMEM),
        pl.BlockSpec(memory_space=pltpu.VMEM),
    ]
    scratch = (
        [pltpu.VMEM((LN_R, HIDDEN), jnp.float32) for _ in range(2 * LN_NB)]
        + [pltpu.SemaphoreType.DMA for _ in range(2 * LN_NB)]
    )
    n_total = sum(CHUNK_SIZES)
    if out_buf is None:
        return pl.pallas_call(
            body,
            in_specs=data_specs,
            out_specs=pl.BlockSpec(memory_space=pl.ANY),
            out_shape=jax.ShapeDtypeStruct((n_total, HIDDEN), jnp.float32),
            scratch_shapes=scratch,
        )(gathered, pos_table, gamma2, beta2)
    return pl.pallas_call(
        lambda alias_ref, *a: body(*a),
        in_specs=[pl.BlockSpec(memory_space=pl.ANY)] + data_specs,
        out_specs=pl.BlockSpec(memory_space=pl.ANY),
        out_shape=jax.ShapeDtypeStruct((n_total, HIDDEN), jnp.float32),
        input_output_aliases={0: 0},
        scratch_shapes=scratch,
    )(out_buf, gathered, pos_table, gamma2, beta2)


def kernel(input_ids, word_table, pos_table, gamma, beta):
    B, S = input_ids.shape
    n = B * S
    ids_flat = input_ids.reshape(n).astype(jnp.int32)
    gamma2 = gamma.reshape(1, HIDDEN)
    beta2 = beta.reshape(1, HIDDEN)

    offs = [0]
    for sz in CHUNK_SIZES:
        offs.append(offs[-1] + sz)
    assert offs[-1] == n

    gathered = [
        _sc_gather(ids_flat, word_table, offs[k], sz)
        for k, sz in enumerate(CHUNK_SIZES)
    ]

    out = None
    for k in range(len(CHUNK_SIZES)):
        out = _tc_layernorm_chunk(out, gathered[k], pos_table, gamma2, beta2,
                                  offs[k])
    return out.reshape(B, S, HIDDEN)
